# baseline (speedup 1.0000x reference)
import functools

import jax
import jax.numpy as jnp
from jax import lax
from jax.experimental import pallas as pl
from jax.experimental.pallas import tpu as pltpu
from jax.experimental.pallas import tpu_sc as plsc

NB = 2048
LANES = 128
CHUNK = 128
NWORKERS = 32


def _sc_degree(row2d, NP):
    NCH = row2d.shape[0]
    nj = NCH // NWORKERS
    rpt = NP // 16
    zeros = jnp.zeros((rpt,), jnp.float32)
    ones = jnp.ones((CHUNK,), jnp.float32)
    mesh = plsc.VectorSubcoreMesh(core_axis_name="c", subcore_axis_name="s")

    @functools.partial(
        pl.kernel, mesh=mesh,
        out_type=jax.ShapeDtypeStruct((2, NP), jnp.float32),
        scratch_types=[
            pltpu.VMEM((NCH // NWORKERS, CHUNK), jnp.int32),
            pltpu.VMEM((CHUNK,), jnp.float32),
            pltpu.VMEM_SHARED((NP,), jnp.float32),
        ],
    )
    def k(r_h, z_h, o_h, deg_h, ridx, onesv, acc):
        c = lax.axis_index("c")
        s = lax.axis_index("s")
        w = s * 2 + c
        pltpu.sync_copy(o_h, onesv)
        pltpu.sync_copy(r_h.at[pl.ds(w * nj, nj)], ridx)
        pltpu.sync_copy(z_h, acc.at[pl.ds(s * rpt, rpt)])
        plsc.subcore_barrier()

        def step(j, carry):
            pltpu.sync_copy(onesv, acc.at[ridx.at[j]], add=True)
            return carry

        lax.fori_loop(0, nj, step, 0)
        plsc.subcore_barrier()
        pltpu.sync_copy(acc.at[pl.ds(s * rpt, rpt)],
                        deg_h.at[c, pl.ds(s * rpt, rpt)])

    return k(row2d, zeros, ones)


NBUF = 2


def _sc_spmm(t_flat, colg3, row2d, G, NP):
    EP = row2d.size
    nj = EP // CHUNK // NWORKERS
    rpt = NP // 16
    zeros = jnp.zeros((rpt, LANES), jnp.float32)
    colg = colg3.reshape(G, EP)
    rowf = row2d.reshape(EP)
    mesh = plsc.VectorSubcoreMesh(core_axis_name="c", subcore_axis_name="s")

    @functools.partial(
        pl.kernel, mesh=mesh,
        out_type=jax.ShapeDtypeStruct((2, G, NP, LANES), jnp.float32),
        scratch_types=[
            [pltpu.VMEM((CHUNK,), jnp.int32) for _ in range(4)],
            [pltpu.VMEM((CHUNK,), jnp.int32) for _ in range(4)],
            [pltpu.VMEM((CHUNK, LANES), jnp.float32) for _ in range(2)],
            pltpu.VMEM_SHARED((NP, LANES), jnp.float32),
            [pltpu.SemaphoreType.DMA for _ in range(4)],
            [pltpu.SemaphoreType.DMA for _ in range(4)],
            [pltpu.SemaphoreType.DMA for _ in range(2)],
            [pltpu.SemaphoreType.DMA for _ in range(2)],
        ],
    )
    def k(t_h, cg_h, r_h, z_h, y_h, ci, ri, rb, acc, sc_, sr, sg, ss):
        c = lax.axis_index("c")
        s = lax.axis_index("s")
        w = s * 2 + c

        def fire_idx(g, j, q):
            cb = (w + NWORKERS * j) * CHUNK
            pltpu.async_copy(cg_h.at[g, pl.ds(cb, CHUNK)], ci[q], sc_[q])
            pltpu.async_copy(r_h.at[pl.ds(cb, CHUNK)], ri[q], sr[q])

        def wait_idx(g, q):
            pltpu.make_async_copy(cg_h.at[g, pl.ds(0, CHUNK)], ci[q],
                                  sc_[q]).wait()
            pltpu.make_async_copy(r_h.at[pl.ds(0, CHUNK)], ri[q],
                                  sr[q]).wait()

        def wait_gather(m):
            pltpu.make_async_copy(t_h.at[ci[0]], rb[m], sg[m]).wait()

        def wait_scatter(m, q):
            pltpu.make_async_copy(rb[m], acc.at[ri[q]], ss[m]).wait()

        for g in range(G):
            pltpu.sync_copy(z_h, acc.at[pl.ds(s * rpt, rpt)])
            plsc.subcore_barrier()

            fire_idx(g, 0, 0)
            fire_idx(g, 1, 1)
            fire_idx(g, 2, 2)
            wait_idx(g, 0)
            pltpu.async_copy(t_h.at[ci[0]], rb[0], sg[0])

            def four(jj, carry):
                for k in range(4):
                    j2 = jj * 4 + k
                    m = k % 2
                    mn = 1 - m
                    q = k

                    wait_gather(m)
                    pltpu.async_copy(rb[m], acc.at[ri[q]], ss[m], add=True)

                    @pl.when(j2 >= 1)
                    def _():
                        wait_scatter(mn, (k - 1) % 4)

                    @pl.when(j2 + 1 < nj)
                    def _():
                        wait_idx(g, (k + 1) % 4)
                        pltpu.async_copy(t_h.at[ci[(k + 1) % 4]], rb[mn],
                                         sg[mn])

                    @pl.when(j2 + 3 < nj)
                    def _():
                        fire_idx(g, j2 + 3, (k + 3) % 4)
                return carry

            lax.fori_loop(0, nj // 4, four, 0)
            wait_scatter((nj - 1) % 2, (nj - 1) % 4)
            plsc.subcore_barrier()
            pltpu.sync_copy(acc.at[pl.ds(s * rpt, rpt)],
                            y_h.at[c, g, pl.ds(s * rpt, rpt)])
            plsc.subcore_barrier()

    return k(t_flat, colg, rowf, zeros)


def _tc_dinv(deg_parts, NP):
    rows = NP // LANES

    def body(d_ref, o_ref):
        deg = d_ref[0:rows, :] + d_ref[rows:2 * rows, :]
        o_ref[...] = jnp.where(deg > 0.5, lax.rsqrt(jnp.maximum(deg, 1.0)), 0.0)

    out = pl.pallas_call(
        body,
        out_shape=jax.ShapeDtypeStruct((rows, LANES), jnp.float32),
    )(deg_parts.reshape(2 * rows, LANES))
    return out.reshape(NP, 1)


def _tc_l0_pre(x_p, d, W1, W2, W02, b0, NP):
    B = x_p.shape[0]
    nsteps = NP // NB
    co = W1.shape[1]
    btile = jnp.tile(b0, (B,))[None, :]

    def body(x_ref, d_ref, w1_ref, w2_ref, w02_ref, b_ref,
             u1_ref, u2d_ref, hw_ref):
        dv = d_ref[...]

        def mm(w):
            return jnp.concatenate(
                [jnp.dot(x_ref[b], w, preferred_element_type=jnp.float32)
                 for b in range(B)], axis=1)

        u1_ref[...] = mm(w1_ref[...])
        u2d_ref[...] = dv * mm(w2_ref[...])
        hw_ref[...] = mm(w02_ref[...]) + b_ref[...]

    outs = pl.pallas_call(
        body,
        grid=(nsteps,),
        in_specs=[
            pl.BlockSpec((B, NB, 128), lambda n: (0, n, 0)),
            pl.BlockSpec((NB, 1), lambda n: (n, 0)),
            pl.BlockSpec((128, co), lambda n: (0, 0)),
            pl.BlockSpec((128, co), lambda n: (0, 0)),
            pl.BlockSpec((128, co), lambda n: (0, 0)),
            pl.BlockSpec((1, B * co), lambda n: (0, 0)),
        ],
        out_specs=[
            pl.BlockSpec((NB, B * co), lambda n: (n, 0)),
            pl.BlockSpec((NB, B * co), lambda n: (n, 0)),
            pl.BlockSpec((NB, B * co), lambda n: (n, 0)),
        ],
        out_shape=[jax.ShapeDtypeStruct((NP, B * co), jnp.float32)] * 3,
    )(x_p, d, W1, W2, W02, btile)
    return outs


def _tc_l0_mid(u1, a1, d, NP):
    nsteps = NP // NB

    def body(u_ref, a_ref, d_ref, o_ref):
        dv = d_ref[...]
        asum = a_ref[0, 0] + a_ref[1, 0]
        o_ref[...] = dv * u_ref[...] - 2.0 * (dv * dv) * asum

    return pl.pallas_call(
        body,
        grid=(nsteps,),
        in_specs=[
            pl.BlockSpec((NB, 128), lambda n: (n, 0)),
            pl.BlockSpec((2, 1, NB, 128), lambda n: (0, 0, n, 0)),
            pl.BlockSpec((NB, 1), lambda n: (n, 0)),
        ],
        out_specs=pl.BlockSpec((NB, 128), lambda n: (n, 0)),
        out_shape=jax.ShapeDtypeStruct((NP, 128), jnp.float32),
    )(u1, a1, d)


def _tc_l0_post(hw, a2, d, NP):
    nsteps = NP // NB

    def body(hw_ref, a_ref, d_ref, o_ref):
        asum = a_ref[0, 0] + a_ref[1, 0]
        o_ref[...] = jnp.maximum(hw_ref[...] - d_ref[...] * asum, 0.0)

    return pl.pallas_call(
        body,
        grid=(nsteps,),
        in_specs=[
            pl.BlockSpec((NB, 128), lambda n: (n, 0)),
            pl.BlockSpec((2, 1, NB, 128), lambda n: (0, 0, n, 0)),
            pl.BlockSpec((NB, 1), lambda n: (n, 0)),
        ],
        out_specs=pl.BlockSpec((NB, 128), lambda n: (n, 0)),
        out_shape=jax.ShapeDtypeStruct((NP, 128), jnp.float32),
    )(hw, a2, d)


def _tc_scale(h, d, NP):
    W = h.shape[1]
    nsteps = NP // NB

    def body(h_ref, d_ref, o_ref):
        o_ref[...] = d_ref[...] * h_ref[...]

    return pl.pallas_call(
        body,
        grid=(nsteps,),
        in_specs=[
            pl.BlockSpec((NB, W), lambda n: (n, 0)),
            pl.BlockSpec((NB, 1), lambda n: (n, 0)),
        ],
        out_specs=pl.BlockSpec((NB, W), lambda n: (n, 0)),
        out_shape=jax.ShapeDtypeStruct((NP, W), jnp.float32),
    )(h, d)


def _tc_mid(a1, d, G, NP):
    nsteps = NP // NB

    def body(a_ref, d_ref, o_ref):
        dv = d_ref[...]
        o_ref[0] = -(dv * dv) * (a_ref[0, 0] + a_ref[1, 0])

    return pl.pallas_call(
        body,
        grid=(G, nsteps),
        in_specs=[
            pl.BlockSpec((2, 1, NB, 128), lambda g, n: (0, g, n, 0)),
            pl.BlockSpec((NB, 1), lambda g, n: (n, 0)),
        ],
        out_specs=pl.BlockSpec((1, NB, 128), lambda g, n: (g, n, 0)),
        out_shape=jax.ShapeDtypeStruct((G, NP, 128), jnp.float32),
    )(a1, d)


def _tc_comb(h, a1, a2, d, W02bd, W1bd, W2bd, bbd, G, NP):
    wout = W02bd.shape[1]
    nsteps = NP // NB

    def body(h_ref, a1_ref, a2_ref, d_ref, w0_ref, w1_ref, w2_ref, b_ref, o_ref):
        dv = d_ref[...]
        s1 = dv * (a1_ref[0, 0] + a1_ref[1, 0])
        s2 = dv * (a2_ref[0, 0] + a2_ref[1, 0])
        o = jnp.dot(h_ref[...], w0_ref[...], preferred_element_type=jnp.float32)
        o = o + jnp.dot(s1, w1_ref[...], preferred_element_type=jnp.float32)
        o = o + jnp.dot(s2, w2_ref[...], preferred_element_type=jnp.float32)
        o_ref[...] = jnp.maximum(o + b_ref[...], 0.0)

    return pl.pallas_call(
        body,
        grid=(G, nsteps),
        in_specs=[
            pl.BlockSpec((NB, 128), lambda g, n: (n, g)),
            pl.BlockSpec((2, 1, NB, 128), lambda g, n: (0, g, n, 0)),
            pl.BlockSpec((2, 1, NB, 128), lambda g, n: (0, g, n, 0)),
            pl.BlockSpec((NB, 1), lambda g, n: (n, 0)),
            pl.BlockSpec((128, wout), lambda g, n: (0, 0)),
            pl.BlockSpec((128, wout), lambda g, n: (0, 0)),
            pl.BlockSpec((128, wout), lambda g, n: (0, 0)),
            pl.BlockSpec((1, wout), lambda g, n: (0, 0)),
        ],
        out_specs=pl.BlockSpec((NB, wout), lambda g, n: (n, g)),
        out_shape=jax.ShapeDtypeStruct((NP, G * wout), jnp.float32),
    )(h, a1, a2, d, W02bd, W1bd, W2bd, bbd)


def _tc_head(h5, Wl_p, bl, NP, B):
    nsteps = NP // NB
    C = h5.shape[2]
    OUT = Wl_p.shape[1]

    def body(h_ref, wl_ref, bl_ref, o_ref, acc_ref):
        i = pl.program_id(0)

        @pl.when(i == 0)
        def _():
            acc_ref[...] = jnp.zeros_like(acc_ref)

        hm = jnp.sum(h_ref[...], axis=2) * (1.0 / C)
        acc_ref[...] += lax.dot_general(
            hm, wl_ref[...], (((0,), (0,)), ((), ())),
            preferred_element_type=jnp.float32)

        @pl.when(i == nsteps - 1)
        def _():
            lg = acc_ref[...] + bl_ref[...]
            m = jnp.max(lg, axis=1, keepdims=True)
            lse = jnp.log(jnp.sum(jnp.exp(lg - m), axis=1, keepdims=True)) + m
            o_ref[...] = lg - lse

    return pl.pallas_call(
        body,
        grid=(nsteps,),
        in_specs=[
            pl.BlockSpec((NB, B, C), lambda n: (n, 0, 0)),
            pl.BlockSpec((NB, OUT), lambda n: (n, 0)),
            pl.BlockSpec((B, OUT), lambda n: (0, 0)),
        ],
        out_specs=pl.BlockSpec((B, OUT), lambda n: (0, 0)),
        out_shape=jax.ShapeDtypeStruct((B, OUT), jnp.float32),
        scratch_shapes=[pltpu.VMEM((B, OUT), jnp.float32)],
    )(h5, Wl_p, jnp.broadcast_to(bl[None, :], (B, OUT)))


def kernel(x, edge_index, W0, b0, W1, b1, W2, b2, W3, b3, W4, b4, Wl, bl):
    B, N, CIN0 = x.shape
    NP = 10240
    row, col = edge_index[0], edge_index[1]
    E = row.shape[0]

    nch = -(-(E // CHUNK) // (NWORKERS * 8)) * NWORKERS * 8
    ep = nch * CHUNK
    pad_i = jnp.arange(ep - E, dtype=jnp.int32)
    row_p = jnp.concatenate(
        [row, N + pad_i % (NP - N)]).reshape(nch, CHUNK)
    col_p = jnp.concatenate([col, pad_i % N])

    colg1 = {}
    colg2 = {}
    for G in (1, 2, 4):
        if G == 1:
            colg1[G] = colg2[G] = col_p.reshape(1, nch, CHUNK)
        else:
            g = jnp.arange(G, dtype=jnp.int32)[:, None]
            colg1[G] = (G * col_p[None, :] + g).reshape(G, nch, CHUNK)
            colg2[G] = (g * NP + col_p[None, :]).reshape(G, nch, CHUNK)

    deg_parts = _sc_degree(row_p, NP)
    d = _tc_dinv(deg_parts, NP)

    x_p = jnp.pad(x, ((0, 0), (0, NP - N), (0, 0)))

    u1, u2d, hw = _tc_l0_pre(x_p, d, W0[1], W0[2], W0[0] - W0[2], b0, NP)
    a1 = _sc_spmm(u2d, colg1[1], row_p, 1, NP)
    arg2 = _tc_l0_mid(u1, a1, d, NP)
    a2 = _sc_spmm(arg2, colg2[1], row_p, 1, NP)
    h = _tc_l0_post(hw, a2, d, NP)

    for Wt, bt in ((W1, b1), (W2, b2), (W3, b3), (W4, b4)):
        cin, cout = Wt.shape[1], Wt.shape[2]
        G = (B * cin) // 128
        perg = 128 // cin
        eye = jnp.eye(perg, dtype=jnp.float32)
        W02bd = jnp.kron(eye, Wt[0] - Wt[2])
        W1bd = jnp.kron(eye, -Wt[1])
        W2bd = jnp.kron(eye, -2.0 * Wt[2])
        bbd = jnp.tile(bt, (perg,))[None, :]

        hd = _tc_scale(h, d, NP)
        a1 = _sc_spmm(hd.reshape(NP * G, 128), colg1[G], row_p, G, NP)
        arg2 = _tc_mid(a1, d, G, NP)
        a2 = _sc_spmm(arg2.reshape(G * NP, 128), colg2[G], row_p, G, NP)
        h = _tc_comb(h, a1, a2, d, W02bd, W1bd, W2bd, bbd, G, NP)

    Wl_p = jnp.pad(Wl, ((0, NP - N), (0, 0)))
    return _tc_head(h.reshape(NP, B, 64), Wl_p, bl, NP, B)


# speedup vs baseline: 1.1858x; 1.1858x over previous
import functools

import jax
import jax.numpy as jnp
from jax import lax
from jax.experimental import pallas as pl
from jax.experimental.pallas import tpu as pltpu
from jax.experimental.pallas import tpu_sc as plsc

NB = 2048
LANES = 128
CHUNK = 128
NWORKERS = 32


def _sc_degree(row2d, NP):
    NCH = row2d.shape[0]
    nj = NCH // NWORKERS
    rpt = NP // 16
    zeros = jnp.zeros((rpt,), jnp.float32)
    ones = jnp.ones((CHUNK,), jnp.float32)
    mesh = plsc.VectorSubcoreMesh(core_axis_name="c", subcore_axis_name="s")

    @functools.partial(
        pl.kernel, mesh=mesh,
        out_type=jax.ShapeDtypeStruct((2, NP), jnp.float32),
        scratch_types=[
            pltpu.VMEM((NCH // NWORKERS, CHUNK), jnp.int32),
            pltpu.VMEM((CHUNK,), jnp.float32),
            pltpu.VMEM_SHARED((NP,), jnp.float32),
        ],
    )
    def k(r_h, z_h, o_h, deg_h, ridx, onesv, acc):
        c = lax.axis_index("c")
        s = lax.axis_index("s")
        w = s * 2 + c
        pltpu.sync_copy(o_h, onesv)
        pltpu.sync_copy(r_h.at[pl.ds(w * nj, nj)], ridx)
        pltpu.sync_copy(z_h, acc.at[pl.ds(s * rpt, rpt)])
        plsc.subcore_barrier()

        def step(j, carry):
            pltpu.sync_copy(onesv, acc.at[ridx.at[j]], add=True)
            return carry

        lax.fori_loop(0, nj, step, 0)
        plsc.subcore_barrier()
        pltpu.sync_copy(acc.at[pl.ds(s * rpt, rpt)],
                        deg_h.at[c, pl.ds(s * rpt, rpt)])

    return k(row2d, zeros, ones)


NBUF = 2


def _sc_spmm(t_flat, colg3, row2d, G, NP):
    EP = row2d.size
    nj = EP // CHUNK // NWORKERS
    rpt = NP // 16
    zeros = jnp.zeros((rpt, LANES), jnp.float32)
    colg = colg3.reshape(G, EP)
    rowf = row2d.reshape(EP)
    mesh = plsc.VectorSubcoreMesh(core_axis_name="c", subcore_axis_name="s")

    @functools.partial(
        pl.kernel, mesh=mesh,
        out_type=jax.ShapeDtypeStruct((2, G, NP, LANES), jnp.float32),
        scratch_types=[
            [pltpu.VMEM((CHUNK,), jnp.int32) for _ in range(4)],
            [pltpu.VMEM((CHUNK,), jnp.int32) for _ in range(4)],
            [pltpu.VMEM((CHUNK, LANES), jnp.float32) for _ in range(2)],
            pltpu.VMEM_SHARED((NP, LANES), jnp.float32),
            [pltpu.SemaphoreType.DMA for _ in range(4)],
            [pltpu.SemaphoreType.DMA for _ in range(4)],
            [pltpu.SemaphoreType.DMA for _ in range(2)],
            [pltpu.SemaphoreType.DMA for _ in range(2)],
        ],
    )
    def k(t_h, cg_h, r_h, z_h, y_h, ci, ri, rb, acc, sc_, sr, sg, ss):
        c = lax.axis_index("c")
        s = lax.axis_index("s")
        w = s * 2 + c

        def fire_idx(g, j, q):
            cb = (w + NWORKERS * j) * CHUNK
            pltpu.async_copy(cg_h.at[g, pl.ds(cb, CHUNK)], ci[q], sc_[q])
            pltpu.async_copy(r_h.at[pl.ds(cb, CHUNK)], ri[q], sr[q])

        def wait_idx(g, q):
            pltpu.make_async_copy(cg_h.at[g, pl.ds(0, CHUNK)], ci[q],
                                  sc_[q]).wait()
            pltpu.make_async_copy(r_h.at[pl.ds(0, CHUNK)], ri[q],
                                  sr[q]).wait()

        def wait_gather(m):
            pltpu.make_async_copy(t_h.at[ci[0]], rb[m], sg[m]).wait()

        def wait_scatter(m, q):
            pltpu.make_async_copy(rb[m], acc.at[ri[q]], ss[m]).wait()

        for g in range(G):
            pltpu.sync_copy(z_h, acc.at[pl.ds(s * rpt, rpt)])
            plsc.subcore_barrier()

            fire_idx(g, 0, 0)
            fire_idx(g, 1, 1)
            fire_idx(g, 2, 2)
            wait_idx(g, 0)
            pltpu.async_copy(t_h.at[ci[0]], rb[0], sg[0])

            def four(jj, carry):
                for k in range(4):
                    j2 = jj * 4 + k
                    m = k % 2
                    mn = 1 - m

                    @pl.when(j2 + 1 < nj)
                    def _():
                        wait_idx(g, (k + 1) % 4)
                        pltpu.async_copy(t_h.at[ci[(k + 1) % 4]], rb[mn],
                                         sg[mn])

                    wait_gather(m)
                    pltpu.sync_copy(rb[m], acc.at[ri[k]], add=True)

                    @pl.when(j2 + 3 < nj)
                    def _():
                        fire_idx(g, j2 + 3, (k + 3) % 4)
                return carry

            lax.fori_loop(0, nj // 4, four, 0)
            plsc.subcore_barrier()
            pltpu.sync_copy(acc.at[pl.ds(s * rpt, rpt)],
                            y_h.at[c, g, pl.ds(s * rpt, rpt)])
            plsc.subcore_barrier()

    return k(t_flat, colg, rowf, zeros)


def _tc_dinv(deg_parts, NP):
    rows = NP // LANES

    def body(d_ref, o_ref):
        deg = d_ref[0:rows, :] + d_ref[rows:2 * rows, :]
        o_ref[...] = jnp.where(deg > 0.5, lax.rsqrt(jnp.maximum(deg, 1.0)), 0.0)

    out = pl.pallas_call(
        body,
        out_shape=jax.ShapeDtypeStruct((rows, LANES), jnp.float32),
    )(deg_parts.reshape(2 * rows, LANES))
    return out.reshape(NP, 1)


def _tc_l0_pre(x_p, d, W1, W2, W02, b0, NP):
    B = x_p.shape[0]
    nsteps = NP // NB
    co = W1.shape[1]
    btile = jnp.tile(b0, (B,))[None, :]

    def body(x_ref, d_ref, w1_ref, w2_ref, w02_ref, b_ref,
             u1_ref, u2d_ref, hw_ref):
        dv = d_ref[...]

        def mm(w):
            return jnp.concatenate(
                [jnp.dot(x_ref[b], w, preferred_element_type=jnp.float32)
                 for b in range(B)], axis=1)

        u1_ref[...] = mm(w1_ref[...])
        u2d_ref[...] = dv * mm(w2_ref[...])
        hw_ref[...] = mm(w02_ref[...]) + b_ref[...]

    outs = pl.pallas_call(
        body,
        grid=(nsteps,),
        in_specs=[
            pl.BlockSpec((B, NB, 128), lambda n: (0, n, 0)),
            pl.BlockSpec((NB, 1), lambda n: (n, 0)),
            pl.BlockSpec((128, co), lambda n: (0, 0)),
            pl.BlockSpec((128, co), lambda n: (0, 0)),
            pl.BlockSpec((128, co), lambda n: (0, 0)),
            pl.BlockSpec((1, B * co), lambda n: (0, 0)),
        ],
        out_specs=[
            pl.BlockSpec((NB, B * co), lambda n: (n, 0)),
            pl.BlockSpec((NB, B * co), lambda n: (n, 0)),
            pl.BlockSpec((NB, B * co), lambda n: (n, 0)),
        ],
        out_shape=[jax.ShapeDtypeStruct((NP, B * co), jnp.float32)] * 3,
    )(x_p, d, W1, W2, W02, btile)
    return outs


def _tc_l0_mid(u1, a1, d, NP):
    nsteps = NP // NB

    def body(u_ref, a_ref, d_ref, o_ref):
        dv = d_ref[...]
        asum = a_ref[0, 0] + a_ref[1, 0]
        o_ref[...] = dv * u_ref[...] - 2.0 * (dv * dv) * asum

    return pl.pallas_call(
        body,
        grid=(nsteps,),
        in_specs=[
            pl.BlockSpec((NB, 128), lambda n: (n, 0)),
            pl.BlockSpec((2, 1, NB, 128), lambda n: (0, 0, n, 0)),
            pl.BlockSpec((NB, 1), lambda n: (n, 0)),
        ],
        out_specs=pl.BlockSpec((NB, 128), lambda n: (n, 0)),
        out_shape=jax.ShapeDtypeStruct((NP, 128), jnp.float32),
    )(u1, a1, d)


def _tc_l0_post(hw, a2, d, NP):
    nsteps = NP // NB

    def body(hw_ref, a_ref, d_ref, o_ref, od_ref):
        asum = a_ref[0, 0] + a_ref[1, 0]
        dv = d_ref[...]
        h1 = jnp.maximum(hw_ref[...] - dv * asum, 0.0)
        o_ref[...] = h1
        od_ref[...] = dv * h1

    return pl.pallas_call(
        body,
        grid=(nsteps,),
        in_specs=[
            pl.BlockSpec((NB, 128), lambda n: (n, 0)),
            pl.BlockSpec((2, 1, NB, 128), lambda n: (0, 0, n, 0)),
            pl.BlockSpec((NB, 1), lambda n: (n, 0)),
        ],
        out_specs=[pl.BlockSpec((NB, 128), lambda n: (n, 0)),
                   pl.BlockSpec((NB, 128), lambda n: (n, 0))],
        out_shape=[jax.ShapeDtypeStruct((NP, 128), jnp.float32)] * 2,
    )(hw, a2, d)


def _tc_scale(h, d, NP):
    W = h.shape[1]
    nsteps = NP // NB

    def body(h_ref, d_ref, o_ref):
        o_ref[...] = d_ref[...] * h_ref[...]

    return pl.pallas_call(
        body,
        grid=(nsteps,),
        in_specs=[
            pl.BlockSpec((NB, W), lambda n: (n, 0)),
            pl.BlockSpec((NB, 1), lambda n: (n, 0)),
        ],
        out_specs=pl.BlockSpec((NB, W), lambda n: (n, 0)),
        out_shape=jax.ShapeDtypeStruct((NP, W), jnp.float32),
    )(h, d)


def _tc_mid(a1, d, G, NP):
    nsteps = NP // NB

    def body(a_ref, d_ref, o_ref):
        dv = d_ref[...]
        o_ref[0] = -(dv * dv) * (a_ref[0, 0] + a_ref[1, 0])

    return pl.pallas_call(
        body,
        grid=(G, nsteps),
        in_specs=[
            pl.BlockSpec((2, 1, NB, 128), lambda g, n: (0, g, n, 0)),
            pl.BlockSpec((NB, 1), lambda g, n: (n, 0)),
        ],
        out_specs=pl.BlockSpec((1, NB, 128), lambda g, n: (g, n, 0)),
        out_shape=jax.ShapeDtypeStruct((G, NP, 128), jnp.float32),
    )(a1, d)


def _tc_hw(h, W02bd, bbd, G, NP):
    wout = W02bd.shape[1]
    nsteps = NP // NB

    def body(h_ref, w0_ref, b_ref, o_ref):
        o_ref[...] = jnp.dot(h_ref[...], w0_ref[...],
                             preferred_element_type=jnp.float32) + b_ref[...]

    return pl.pallas_call(
        body,
        grid=(G, nsteps),
        in_specs=[
            pl.BlockSpec((NB, 128), lambda g, n: (n, g)),
            pl.BlockSpec((128, wout), lambda g, n: (0, 0)),
            pl.BlockSpec((1, wout), lambda g, n: (0, 0)),
        ],
        out_specs=pl.BlockSpec((NB, wout), lambda g, n: (n, g)),
        out_shape=jax.ShapeDtypeStruct((NP, G * wout), jnp.float32),
    )(h, W02bd, bbd)


def _tc_comb(hw, a1, a2, d, W1bd, W2bd, G, NP):
    wout = W1bd.shape[1]
    nsteps = NP // NB

    def body(hw_ref, a1_ref, a2_ref, d_ref, w1_ref, w2_ref, o_ref, od_ref):
        dv = d_ref[...]
        s1 = dv * (a1_ref[0, 0] + a1_ref[1, 0])
        s2 = dv * (a2_ref[0, 0] + a2_ref[1, 0])
        o = hw_ref[...]
        o = o + jnp.dot(s1, w1_ref[...], preferred_element_type=jnp.float32)
        o = o + jnp.dot(s2, w2_ref[...], preferred_element_type=jnp.float32)
        o = jnp.maximum(o, 0.0)
        o_ref[...] = o
        od_ref[...] = dv * o

    return pl.pallas_call(
        body,
        grid=(G, nsteps),
        in_specs=[
            pl.BlockSpec((NB, wout), lambda g, n: (n, g)),
            pl.BlockSpec((2, 1, NB, 128), lambda g, n: (0, g, n, 0)),
            pl.BlockSpec((2, 1, NB, 128), lambda g, n: (0, g, n, 0)),
            pl.BlockSpec((NB, 1), lambda g, n: (n, 0)),
            pl.BlockSpec((128, wout), lambda g, n: (0, 0)),
            pl.BlockSpec((128, wout), lambda g, n: (0, 0)),
        ],
        out_specs=[pl.BlockSpec((NB, wout), lambda g, n: (n, g)),
                   pl.BlockSpec((NB, wout), lambda g, n: (n, g))],
        out_shape=[jax.ShapeDtypeStruct((NP, G * wout), jnp.float32)] * 2,
    )(hw, a1, a2, d, W1bd, W2bd)


def _tc_head(h5, Wl_p, bl, NP, B):
    nsteps = NP // NB
    C = h5.shape[2]
    OUT = Wl_p.shape[1]

    def body(h_ref, wl_ref, bl_ref, o_ref, acc_ref):
        i = pl.program_id(0)

        @pl.when(i == 0)
        def _():
            acc_ref[...] = jnp.zeros_like(acc_ref)

        hm = jnp.sum(h_ref[...], axis=2) * (1.0 / C)
        acc_ref[...] += lax.dot_general(
            hm, wl_ref[...], (((0,), (0,)), ((), ())),
            preferred_element_type=jnp.float32)

        @pl.when(i == nsteps - 1)
        def _():
            lg = acc_ref[...] + bl_ref[...]
            m = jnp.max(lg, axis=1, keepdims=True)
            lse = jnp.log(jnp.sum(jnp.exp(lg - m), axis=1, keepdims=True)) + m
            o_ref[...] = lg - lse

    return pl.pallas_call(
        body,
        grid=(nsteps,),
        in_specs=[
            pl.BlockSpec((NB, B, C), lambda n: (n, 0, 0)),
            pl.BlockSpec((NB, OUT), lambda n: (n, 0)),
            pl.BlockSpec((B, OUT), lambda n: (0, 0)),
        ],
        out_specs=pl.BlockSpec((B, OUT), lambda n: (0, 0)),
        out_shape=jax.ShapeDtypeStruct((B, OUT), jnp.float32),
        scratch_shapes=[pltpu.VMEM((B, OUT), jnp.float32)],
    )(h5, Wl_p, jnp.broadcast_to(bl[None, :], (B, OUT)))


def kernel(x, edge_index, W0, b0, W1, b1, W2, b2, W3, b3, W4, b4, Wl, bl):
    B, N, CIN0 = x.shape
    NP = 10240
    row, col = edge_index[0], edge_index[1]
    E = row.shape[0]

    nch = -(-(E // CHUNK) // (NWORKERS * 8)) * NWORKERS * 8
    ep = nch * CHUNK
    pad_i = jnp.arange(ep - E, dtype=jnp.int32)
    row_p = jnp.concatenate(
        [row, N + pad_i % (NP - N)]).reshape(nch, CHUNK)
    col_p = jnp.concatenate([col, pad_i % N])

    colg1 = {}
    colg2 = {}
    for G in (1, 2, 4):
        if G == 1:
            colg1[G] = colg2[G] = col_p.reshape(1, nch, CHUNK)
        else:
            g = jnp.arange(G, dtype=jnp.int32)[:, None]
            colg1[G] = (G * col_p[None, :] + g).reshape(G, nch, CHUNK)
            colg2[G] = (g * NP + col_p[None, :]).reshape(G, nch, CHUNK)

    deg_parts = _sc_degree(row_p, NP)
    d = _tc_dinv(deg_parts, NP)

    x_p = jnp.pad(x, ((0, 0), (0, NP - N), (0, 0)))

    u1, u2d, hw = _tc_l0_pre(x_p, d, W0[1], W0[2], W0[0] - W0[2], b0, NP)
    a1 = _sc_spmm(u2d, colg1[1], row_p, 1, NP)
    arg2 = _tc_l0_mid(u1, a1, d, NP)
    a2 = _sc_spmm(arg2, colg2[1], row_p, 1, NP)
    h, hd = _tc_l0_post(hw, a2, d, NP)

    for Wt, bt in ((W1, b1), (W2, b2), (W3, b3), (W4, b4)):
        cin, cout = Wt.shape[1], Wt.shape[2]
        G = (B * cin) // 128
        perg = 128 // cin
        eye = jnp.eye(perg, dtype=jnp.float32)
        W02bd = jnp.kron(eye, Wt[0] - Wt[2])
        W1bd = jnp.kron(eye, -Wt[1])
        W2bd = jnp.kron(eye, -2.0 * Wt[2])
        bbd = jnp.tile(bt, (perg,))[None, :]

        hw = _tc_hw(h, W02bd, bbd, G, NP)
        a1 = _sc_spmm(hd.reshape(NP * G, 128), colg1[G], row_p, G, NP)
        arg2 = _tc_mid(a1, d, G, NP)
        a2 = _sc_spmm(arg2.reshape(G * NP, 128), colg2[G], row_p, G, NP)
        h, hd = _tc_comb(hw, a1, a2, d, W1bd, W2bd, G, NP)

    Wl_p = jnp.pad(Wl, ((0, NP - N), (0, 0)))
    return _tc_head(h.reshape(NP, B, 64), Wl_p, bl, NP, B)


# speedup vs baseline: 1.2696x; 1.0706x over previous
import functools

import jax
import jax.numpy as jnp
from jax import lax
from jax.experimental import pallas as pl
from jax.experimental.pallas import tpu as pltpu
from jax.experimental.pallas import tpu_sc as plsc

NB = 2048
LANES = 128
CHUNK = 128
NWORKERS = 32


def _sc_degree(row2d, NP):
    NCH = row2d.shape[0]
    nj = NCH // NWORKERS
    rpt = NP // 16
    zeros = jnp.zeros((rpt,), jnp.float32)
    ones = jnp.ones((CHUNK,), jnp.float32)
    mesh = plsc.VectorSubcoreMesh(core_axis_name="c", subcore_axis_name="s")

    @functools.partial(
        pl.kernel, mesh=mesh,
        out_type=jax.ShapeDtypeStruct((2, NP), jnp.float32),
        scratch_types=[
            pltpu.VMEM((NCH // NWORKERS, CHUNK), jnp.int32),
            pltpu.VMEM((CHUNK,), jnp.float32),
            pltpu.VMEM_SHARED((NP,), jnp.float32),
        ],
    )
    def k(r_h, z_h, o_h, deg_h, ridx, onesv, acc):
        c = lax.axis_index("c")
        s = lax.axis_index("s")
        w = s * 2 + c
        pltpu.sync_copy(o_h, onesv)
        pltpu.sync_copy(r_h.at[pl.ds(w * nj, nj)], ridx)
        pltpu.sync_copy(z_h, acc.at[pl.ds(s * rpt, rpt)])
        plsc.subcore_barrier()

        def step(j, carry):
            pltpu.sync_copy(onesv, acc.at[ridx.at[j]], add=True)
            return carry

        lax.fori_loop(0, nj, step, 0)
        plsc.subcore_barrier()
        pltpu.sync_copy(acc.at[pl.ds(s * rpt, rpt)],
                        deg_h.at[c, pl.ds(s * rpt, rpt)])

    return k(row2d, zeros, ones)


NBUF = 2


def _sc_spmm(t_flat, colg3, row2d, G, NP, gsplit=False):
    EP = row2d.size
    nw = 16 if gsplit else NWORKERS
    stride = 16 if gsplit else NWORKERS
    nj = EP // CHUNK // nw
    rpt = NP // 16
    zeros = jnp.zeros((rpt, LANES), jnp.float32)
    colg = colg3.reshape(G, EP)
    rowf = row2d.reshape(EP)
    out_sds = (jax.ShapeDtypeStruct((G, NP, LANES), jnp.float32) if gsplit
               else jax.ShapeDtypeStruct((2, G, NP, LANES), jnp.float32))
    mesh = plsc.VectorSubcoreMesh(core_axis_name="c", subcore_axis_name="s")

    @functools.partial(
        pl.kernel, mesh=mesh,
        out_type=out_sds,
        scratch_types=[
            [pltpu.VMEM((CHUNK,), jnp.int32) for _ in range(4)],
            [pltpu.VMEM((CHUNK,), jnp.int32) for _ in range(4)],
            [pltpu.VMEM((CHUNK, LANES), jnp.float32) for _ in range(2)],
            pltpu.VMEM_SHARED((NP, LANES), jnp.float32),
            [pltpu.SemaphoreType.DMA for _ in range(4)],
            [pltpu.SemaphoreType.DMA for _ in range(4)],
            [pltpu.SemaphoreType.DMA for _ in range(2)],
            [pltpu.SemaphoreType.DMA for _ in range(2)],
        ],
    )
    def k(t_h, cg_h, r_h, z_h, y_h, ci, ri, rb, acc, sc_, sr, sg, ss):
        c = lax.axis_index("c")
        s = lax.axis_index("s")
        w = s if gsplit else s * 2 + c

        def fire_idx(g, j, q):
            cb = (w + stride * j) * CHUNK
            pltpu.async_copy(cg_h.at[g, pl.ds(cb, CHUNK)], ci[q], sc_[q])
            pltpu.async_copy(r_h.at[pl.ds(cb, CHUNK)], ri[q], sr[q])

        def wait_idx(g, q):
            pltpu.make_async_copy(cg_h.at[g, pl.ds(0, CHUNK)], ci[q],
                                  sc_[q]).wait()
            pltpu.make_async_copy(r_h.at[pl.ds(0, CHUNK)], ri[q],
                                  sr[q]).wait()

        def wait_gather(m):
            pltpu.make_async_copy(t_h.at[ci[0]], rb[m], sg[m]).wait()

        def wait_scatter(m, q):
            pltpu.make_async_copy(rb[m], acc.at[ri[q]], ss[m]).wait()

        def group_body(g):
            pltpu.sync_copy(z_h, acc.at[pl.ds(s * rpt, rpt)])
            plsc.subcore_barrier()

            fire_idx(g, 0, 0)
            fire_idx(g, 1, 1)
            fire_idx(g, 2, 2)
            wait_idx(g, 0)
            pltpu.async_copy(t_h.at[ci[0]], rb[0], sg[0])

            def four(jj, carry):
                for k in range(4):
                    j2 = jj * 4 + k
                    m = k % 2
                    mn = 1 - m

                    @pl.when(j2 + 1 < nj)
                    def _():
                        wait_idx(g, (k + 1) % 4)
                        pltpu.async_copy(t_h.at[ci[(k + 1) % 4]], rb[mn],
                                         sg[mn])

                    wait_gather(m)
                    pltpu.sync_copy(rb[m], acc.at[ri[k]], add=True)

                    @pl.when(j2 + 3 < nj)
                    def _():
                        fire_idx(g, j2 + 3, (k + 3) % 4)
                return carry

            lax.fori_loop(0, nj // 4, four, 0)
            plsc.subcore_barrier()
            dst = (y_h.at[g, pl.ds(s * rpt, rpt)] if gsplit
                   else y_h.at[c, g, pl.ds(s * rpt, rpt)])
            pltpu.sync_copy(acc.at[pl.ds(s * rpt, rpt)], dst)
            plsc.subcore_barrier()

        for g in range(G):
            if gsplit:
                pl.when(c == (g % 2))(lambda g=g: group_body(g))
            else:
                group_body(g)

    return k(t_flat, colg, rowf, zeros)


def _tc_dinv(deg_parts, NP):
    rows = NP // LANES

    def body(d_ref, o_ref):
        deg = d_ref[0:rows, :] + d_ref[rows:2 * rows, :]
        o_ref[...] = jnp.where(deg > 0.5, lax.rsqrt(jnp.maximum(deg, 1.0)), 0.0)

    out = pl.pallas_call(
        body,
        out_shape=jax.ShapeDtypeStruct((rows, LANES), jnp.float32),
    )(deg_parts.reshape(2 * rows, LANES))
    return out.reshape(NP, 1)


def _tc_l0_pre(x_p, d, W1, W2, W02, b0, NP):
    B = x_p.shape[0]
    nsteps = NP // NB
    co = W1.shape[1]
    btile = jnp.tile(b0, (B,))[None, :]

    def body(x_ref, d_ref, w1_ref, w2_ref, w02_ref, b_ref,
             u1_ref, u2d_ref, hw_ref):
        dv = d_ref[...]

        def mm(w):
            return jnp.concatenate(
                [jnp.dot(x_ref[b], w, preferred_element_type=jnp.float32)
                 for b in range(B)], axis=1)

        u1_ref[...] = mm(w1_ref[...])
        u2d_ref[...] = dv * mm(w2_ref[...])
        hw_ref[...] = mm(w02_ref[...]) + b_ref[...]

    outs = pl.pallas_call(
        body,
        grid=(nsteps,),
        in_specs=[
            pl.BlockSpec((B, NB, 128), lambda n: (0, n, 0)),
            pl.BlockSpec((NB, 1), lambda n: (n, 0)),
            pl.BlockSpec((128, co), lambda n: (0, 0)),
            pl.BlockSpec((128, co), lambda n: (0, 0)),
            pl.BlockSpec((128, co), lambda n: (0, 0)),
            pl.BlockSpec((1, B * co), lambda n: (0, 0)),
        ],
        out_specs=[
            pl.BlockSpec((NB, B * co), lambda n: (n, 0)),
            pl.BlockSpec((NB, B * co), lambda n: (n, 0)),
            pl.BlockSpec((NB, B * co), lambda n: (n, 0)),
        ],
        out_shape=[jax.ShapeDtypeStruct((NP, B * co), jnp.float32)] * 3,
    )(x_p, d, W1, W2, W02, btile)
    return outs


def _tc_l0_mid(u1, a1, d, NP):
    nsteps = NP // NB

    def body(u_ref, a_ref, d_ref, o_ref):
        dv = d_ref[...]
        asum = a_ref[0, 0] + a_ref[1, 0]
        o_ref[...] = dv * u_ref[...] - 2.0 * (dv * dv) * asum

    return pl.pallas_call(
        body,
        grid=(nsteps,),
        in_specs=[
            pl.BlockSpec((NB, 128), lambda n: (n, 0)),
            pl.BlockSpec((2, 1, NB, 128), lambda n: (0, 0, n, 0)),
            pl.BlockSpec((NB, 1), lambda n: (n, 0)),
        ],
        out_specs=pl.BlockSpec((NB, 128), lambda n: (n, 0)),
        out_shape=jax.ShapeDtypeStruct((NP, 128), jnp.float32),
    )(u1, a1, d)


def _tc_l0_post(hw, a2, d, NP):
    nsteps = NP // NB

    def body(hw_ref, a_ref, d_ref, o_ref, od_ref):
        asum = a_ref[0, 0] + a_ref[1, 0]
        dv = d_ref[...]
        h1 = jnp.maximum(hw_ref[...] - dv * asum, 0.0)
        o_ref[...] = h1
        od_ref[...] = dv * h1

    return pl.pallas_call(
        body,
        grid=(nsteps,),
        in_specs=[
            pl.BlockSpec((NB, 128), lambda n: (n, 0)),
            pl.BlockSpec((2, 1, NB, 128), lambda n: (0, 0, n, 0)),
            pl.BlockSpec((NB, 1), lambda n: (n, 0)),
        ],
        out_specs=[pl.BlockSpec((NB, 128), lambda n: (n, 0)),
                   pl.BlockSpec((NB, 128), lambda n: (n, 0))],
        out_shape=[jax.ShapeDtypeStruct((NP, 128), jnp.float32)] * 2,
    )(hw, a2, d)


def _tc_scale(h, d, NP):
    W = h.shape[1]
    nsteps = NP // NB

    def body(h_ref, d_ref, o_ref):
        o_ref[...] = d_ref[...] * h_ref[...]

    return pl.pallas_call(
        body,
        grid=(nsteps,),
        in_specs=[
            pl.BlockSpec((NB, W), lambda n: (n, 0)),
            pl.BlockSpec((NB, 1), lambda n: (n, 0)),
        ],
        out_specs=pl.BlockSpec((NB, W), lambda n: (n, 0)),
        out_shape=jax.ShapeDtypeStruct((NP, W), jnp.float32),
    )(h, d)


def _tc_mid(a1, d, G, NP, gsplit=False):
    nsteps = NP // NB

    def body(a_ref, d_ref, o_ref):
        dv = d_ref[...]
        asum = a_ref[0] if gsplit else a_ref[0, 0] + a_ref[1, 0]
        o_ref[0] = -(dv * dv) * asum

    aspec = (pl.BlockSpec((1, NB, 128), lambda g, n: (g, n, 0)) if gsplit
             else pl.BlockSpec((2, 1, NB, 128), lambda g, n: (0, g, n, 0)))
    return pl.pallas_call(
        body,
        grid=(G, nsteps),
        in_specs=[
            aspec,
            pl.BlockSpec((NB, 1), lambda g, n: (n, 0)),
        ],
        out_specs=pl.BlockSpec((1, NB, 128), lambda g, n: (g, n, 0)),
        out_shape=jax.ShapeDtypeStruct((G, NP, 128), jnp.float32),
    )(a1, d)


def _tc_hw(h, W02bd, bbd, G, NP):
    wout = W02bd.shape[1]
    nsteps = NP // NB

    def body(h_ref, w0_ref, b_ref, o_ref):
        o_ref[...] = jnp.dot(h_ref[...], w0_ref[...],
                             preferred_element_type=jnp.float32) + b_ref[...]

    return pl.pallas_call(
        body,
        grid=(G, nsteps),
        in_specs=[
            pl.BlockSpec((NB, 128), lambda g, n: (n, g)),
            pl.BlockSpec((128, wout), lambda g, n: (0, 0)),
            pl.BlockSpec((1, wout), lambda g, n: (0, 0)),
        ],
        out_specs=pl.BlockSpec((NB, wout), lambda g, n: (n, g)),
        out_shape=jax.ShapeDtypeStruct((NP, G * wout), jnp.float32),
    )(h, W02bd, bbd)


def _tc_comb(hw, a1, a2, d, W1bd, W2bd, G, NP, gsplit=False):
    wout = W1bd.shape[1]
    nsteps = NP // NB
    aspec = (pl.BlockSpec((1, NB, 128), lambda g, n: (g, n, 0)) if gsplit
             else pl.BlockSpec((2, 1, NB, 128), lambda g, n: (0, g, n, 0)))

    def body(hw_ref, a1_ref, a2_ref, d_ref, w1_ref, w2_ref, o_ref, od_ref):
        dv = d_ref[...]
        s1 = dv * (a1_ref[0] if gsplit else a1_ref[0, 0] + a1_ref[1, 0])
        s2 = dv * (a2_ref[0] if gsplit else a2_ref[0, 0] + a2_ref[1, 0])
        o = hw_ref[...]
        o = o + jnp.dot(s1, w1_ref[...], preferred_element_type=jnp.float32)
        o = o + jnp.dot(s2, w2_ref[...], preferred_element_type=jnp.float32)
        o = jnp.maximum(o, 0.0)
        o_ref[...] = o
        od_ref[...] = dv * o

    return pl.pallas_call(
        body,
        grid=(G, nsteps),
        in_specs=[
            pl.BlockSpec((NB, wout), lambda g, n: (n, g)),
            aspec,
            aspec,
            pl.BlockSpec((NB, 1), lambda g, n: (n, 0)),
            pl.BlockSpec((128, wout), lambda g, n: (0, 0)),
            pl.BlockSpec((128, wout), lambda g, n: (0, 0)),
        ],
        out_specs=[pl.BlockSpec((NB, wout), lambda g, n: (n, g)),
                   pl.BlockSpec((NB, wout), lambda g, n: (n, g))],
        out_shape=[jax.ShapeDtypeStruct((NP, G * wout), jnp.float32)] * 2,
    )(hw, a1, a2, d, W1bd, W2bd)


def _tc_head(h5, Wl_p, bl, NP, B):
    nsteps = NP // NB
    C = h5.shape[2]
    OUT = Wl_p.shape[1]

    def body(h_ref, wl_ref, bl_ref, o_ref, acc_ref):
        i = pl.program_id(0)

        @pl.when(i == 0)
        def _():
            acc_ref[...] = jnp.zeros_like(acc_ref)

        hm = jnp.sum(h_ref[...], axis=2) * (1.0 / C)
        acc_ref[...] += lax.dot_general(
            hm, wl_ref[...], (((0,), (0,)), ((), ())),
            preferred_element_type=jnp.float32)

        @pl.when(i == nsteps - 1)
        def _():
            lg = acc_ref[...] + bl_ref[...]
            m = jnp.max(lg, axis=1, keepdims=True)
            lse = jnp.log(jnp.sum(jnp.exp(lg - m), axis=1, keepdims=True)) + m
            o_ref[...] = lg - lse

    return pl.pallas_call(
        body,
        grid=(nsteps,),
        in_specs=[
            pl.BlockSpec((NB, B, C), lambda n: (n, 0, 0)),
            pl.BlockSpec((NB, OUT), lambda n: (n, 0)),
            pl.BlockSpec((B, OUT), lambda n: (0, 0)),
        ],
        out_specs=pl.BlockSpec((B, OUT), lambda n: (0, 0)),
        out_shape=jax.ShapeDtypeStruct((B, OUT), jnp.float32),
        scratch_shapes=[pltpu.VMEM((B, OUT), jnp.float32)],
    )(h5, Wl_p, jnp.broadcast_to(bl[None, :], (B, OUT)))


def kernel(x, edge_index, W0, b0, W1, b1, W2, b2, W3, b3, W4, b4, Wl, bl):
    B, N, CIN0 = x.shape
    NP = 10240
    row, col = edge_index[0], edge_index[1]
    E = row.shape[0]

    nch = -(-(E // CHUNK) // (NWORKERS * 8)) * NWORKERS * 8
    ep = nch * CHUNK
    pad_i = jnp.arange(ep - E, dtype=jnp.int32)
    row_p = jnp.concatenate(
        [row, N + pad_i % (NP - N)]).reshape(nch, CHUNK)
    col_p = jnp.concatenate([col, pad_i % N])

    colg1 = {}
    colg2 = {}
    for G in (1, 2, 4):
        if G == 1:
            colg1[G] = colg2[G] = col_p.reshape(1, nch, CHUNK)
        else:
            g = jnp.arange(G, dtype=jnp.int32)[:, None]
            colg1[G] = (G * col_p[None, :] + g).reshape(G, nch, CHUNK)
            colg2[G] = (g * NP + col_p[None, :]).reshape(G, nch, CHUNK)

    deg_parts = _sc_degree(row_p, NP)
    d = _tc_dinv(deg_parts, NP)

    x_p = jnp.pad(x, ((0, 0), (0, NP - N), (0, 0)))

    u1, u2d, hw = _tc_l0_pre(x_p, d, W0[1], W0[2], W0[0] - W0[2], b0, NP)
    a1 = _sc_spmm(u2d, colg1[1], row_p, 1, NP)
    arg2 = _tc_l0_mid(u1, a1, d, NP)
    a2 = _sc_spmm(arg2, colg2[1], row_p, 1, NP)
    h, hd = _tc_l0_post(hw, a2, d, NP)

    for Wt, bt in ((W1, b1), (W2, b2), (W3, b3), (W4, b4)):
        cin, cout = Wt.shape[1], Wt.shape[2]
        G = (B * cin) // 128
        perg = 128 // cin
        eye = jnp.eye(perg, dtype=jnp.float32)
        W02bd = jnp.kron(eye, Wt[0] - Wt[2])
        W1bd = jnp.kron(eye, -Wt[1])
        W2bd = jnp.kron(eye, -2.0 * Wt[2])
        bbd = jnp.tile(bt, (perg,))[None, :]

        gs = G >= 2
        hw = _tc_hw(h, W02bd, bbd, G, NP)
        a1 = _sc_spmm(hd.reshape(NP * G, 128), colg1[G], row_p, G, NP, gs)
        arg2 = _tc_mid(a1, d, G, NP, gs)
        a2 = _sc_spmm(arg2.reshape(G * NP, 128), colg2[G], row_p, G, NP, gs)
        h, hd = _tc_comb(hw, a1, a2, d, W1bd, W2bd, G, NP, gs)

    Wl_p = jnp.pad(Wl, ((0, NP - N), (0, 0)))
    return _tc_head(h.reshape(NP, B, 64), Wl_p, bl, NP, B)
